# Initial kernel scaffold; baseline (speedup 1.0000x reference)
import jax
import jax.numpy as jnp
from jax.experimental import pallas as pl


def kernel(x, node_type, node_attr, W_net, b_net, W_dev, b_dev, pin_table, W_x, b_x):
    raise NotImplementedError("write your pallas kernel here")


# speedup vs baseline: 2.6568x; 2.6568x over previous
import functools

import jax
import jax.numpy as jnp
from jax.experimental import pallas as pl

N_ROWS = 100000
DIM_IN = 512
DIM_EMB = 512
DIM_PE = 64
DIM_H = DIM_EMB - DIM_PE
ATTR = 17
BLOCK = 2000


def _fused_kernel(x_ref, type_ref, attr_ref, w1_ref, b1_ref,
                  wnet_ref, wdev_ref, wpin_ref, bnet_ref, bdev_ref,
                  out_ref):
    x = x_ref[...]
    attr = attr_ref[...]
    t = type_ref[...]

    main = jnp.dot(x, w1_ref[...], preferred_element_type=jnp.float32)
    main = main + b1_ref[...]

    emb_net = jnp.dot(attr, wnet_ref[...], preferred_element_type=jnp.float32)
    emb_dev = jnp.dot(attr, wdev_ref[...], preferred_element_type=jnp.float32)
    idx = jnp.clip(attr[:, 0].astype(jnp.int32), 0, ATTR - 1)
    onehot = (idx[:, None] ==
              jax.lax.broadcasted_iota(jnp.int32, (x.shape[0], ATTR), 1)
              ).astype(jnp.float32)
    emb_pin = jnp.dot(onehot, wpin_ref[...], preferred_element_type=jnp.float32)

    net_m = (t == 0).astype(jnp.float32)
    dev_m = (t == 1).astype(jnp.float32)
    pin_m = (t == 2).astype(jnp.float32)
    emb = (net_m * (emb_net + bnet_ref[...])
           + dev_m * (emb_dev + bdev_ref[...])
           + pin_m * emb_pin)

    out_ref[:, :384] = main[:, :384]
    out_ref[:, 384:] = main[:, 384:] + emb


def kernel(x, node_type, node_attr, W_net, b_net, W_dev, b_dev, pin_table,
           W_x, b_x):
    f32 = jnp.float32
    w1 = jnp.concatenate([W_x.T, jnp.zeros((DIM_IN, DIM_PE), f32)], axis=1)
    b1 = jnp.concatenate([b_x, jnp.zeros((DIM_PE,), f32)])[None, :]
    zpad = jnp.zeros((ATTR, DIM_PE), f32)
    wnet = jnp.concatenate([zpad, W_net.T], axis=1)
    wdev = jnp.concatenate([zpad, W_dev.T], axis=1)
    wpin = jnp.concatenate([zpad, pin_table], axis=1)
    zb = jnp.zeros((DIM_PE,), f32)
    bnet = jnp.concatenate([zb, b_net])[None, :]
    bdev = jnp.concatenate([zb, b_dev])[None, :]

    node_type2 = node_type.reshape(N_ROWS, 1)

    grid = (N_ROWS // BLOCK,)
    const = lambda i: (0, 0)
    out = pl.pallas_call(
        _fused_kernel,
        grid=grid,
        in_specs=[
            pl.BlockSpec((BLOCK, DIM_IN), lambda i: (i, 0)),
            pl.BlockSpec((BLOCK, 1), lambda i: (i, 0)),
            pl.BlockSpec((BLOCK, ATTR), lambda i: (i, 0)),
            pl.BlockSpec((DIM_IN, DIM_EMB), const),
            pl.BlockSpec((1, DIM_EMB), const),
            pl.BlockSpec((ATTR, 2 * DIM_PE), const),
            pl.BlockSpec((ATTR, 2 * DIM_PE), const),
            pl.BlockSpec((ATTR, 2 * DIM_PE), const),
            pl.BlockSpec((1, 2 * DIM_PE), const),
            pl.BlockSpec((1, 2 * DIM_PE), const),
        ],
        out_specs=pl.BlockSpec((BLOCK, DIM_EMB), lambda i: (i, 0)),
        out_shape=jax.ShapeDtypeStruct((N_ROWS, DIM_EMB), f32),
    )(x, node_type2, node_attr, w1, b1, wnet, wdev, wpin, bnet, bdev)
    return out
